# Initial kernel scaffold; baseline (speedup 1.0000x reference)
import jax
import jax.numpy as jnp
from jax.experimental import pallas as pl


def kernel(z, pe):
    raise NotImplementedError("write your pallas kernel here")


# speedup vs baseline: 2.7288x; 2.7288x over previous
import jax
import jax.numpy as jnp
from jax.experimental import pallas as pl

SEQ_LEN = 2048
D_MODEL = 1024


def _bcast_kernel(pe_ref, out_ref):
    blk = pe_ref[...]
    out_ref[...] = jnp.broadcast_to(blk[:, None, :], out_ref.shape)


def kernel(z, pe):
    n = z.shape[1]
    s_blk = 256
    return pl.pallas_call(
        _bcast_kernel,
        grid=(SEQ_LEN // s_blk,),
        in_specs=[pl.BlockSpec((s_blk, D_MODEL), lambda i: (i, 0))],
        out_specs=pl.BlockSpec((s_blk, n, D_MODEL), lambda i: (i, 0, 0)),
        out_shape=jax.ShapeDtypeStruct((SEQ_LEN, n, D_MODEL), z.dtype),
    )(pe)
